# Initial kernel scaffold; baseline (speedup 1.0000x reference)
import functools

import jax
import jax.numpy as jnp
from jax import lax
from jax.experimental import pallas as pl
from jax.experimental.pallas import tpu as pltpu
from jax.experimental.pallas import tpu_sc as plsc

_SCALING = 16 / 16

_R = 16
_V = 100000
_D = 128
_B = 4096
_L = 200
_NTOK = _B * _L

_NC = 2
_NS = 16
_NW = _NC * _NS
_RPC = 128
_NCHUNK = _NTOK // (_NW * _RPC)

_VBLK = 2000


def _table_body(a_ref, b_ref, o_ref):
    o_ref[...] = lax.dot_general(
        a_ref[...], b_ref[...],
        dimension_numbers=(((0,), (1,)), ((), ())),
        preferred_element_type=jnp.float32,
    ) * _SCALING


def _build_table(lora_A, lora_B):
    return pl.pallas_call(
        _table_body,
        grid=(_V // _VBLK,),
        in_specs=[
            pl.BlockSpec((_R, _VBLK), lambda i: (0, i)),
            pl.BlockSpec((_D, _R), lambda i: (0, 0)),
        ],
        out_specs=pl.BlockSpec((_VBLK, _D), lambda i: (i, 0)),
        out_shape=jax.ShapeDtypeStruct((_V, _D), jnp.float32),
    )(lora_A, lora_B)


_sc_mesh = plsc.VectorSubcoreMesh(core_axis_name="c", subcore_axis_name="s")


@functools.partial(
    pl.kernel,
    out_type=jax.ShapeDtypeStruct((_NW, _NCHUNK, _RPC, _D), jnp.float32),
    mesh=_sc_mesh,
    scratch_types=[
        pltpu.VMEM((_NCHUNK, _RPC), jnp.int32),
        pltpu.VMEM((_RPC, _D), jnp.float32),
        pltpu.SemaphoreType.DMA,
    ],
)
def _gather(table_hbm, idx_hbm, out_hbm, idx_v, rows_v, sem):
    wid = lax.axis_index("s") * _NC + lax.axis_index("c")
    pltpu.sync_copy(idx_hbm.at[wid], idx_v)

    def body(j, carry):
        pltpu.async_copy(table_hbm.at[idx_v.at[j]], rows_v, sem).wait()
        pltpu.sync_copy(rows_v, out_hbm.at[wid, j])
        return carry

    lax.fori_loop(0, _NCHUNK, body, 0)


def kernel(x, lora_A, lora_B):
    idx = x.astype(jnp.int32).reshape(_NW, _NCHUNK, _RPC)
    table = _build_table(lora_A, lora_B)
    out = _gather(table, idx)
    return out.reshape(_B, _L, _D)


# speedup vs baseline: 6.2207x; 6.2207x over previous
import functools

import jax
import jax.numpy as jnp
from jax import lax
from jax.experimental import pallas as pl
from jax.experimental.pallas import tpu as pltpu
from jax.experimental.pallas import tpu_sc as plsc

_SCALING = 16 / 16

_R = 16
_V = 100000
_D = 128
_B = 4096
_L = 200
_NTOK = _B * _L

_NC = 2
_NS = 16
_NW = _NC * _NS
_RPC = 128
_NCHUNK = _NTOK // (_NW * _RPC)

_VBLK = 2000


def _table_body(a_ref, b_ref, o_ref):
    o_ref[...] = lax.dot_general(
        a_ref[...], b_ref[...],
        dimension_numbers=(((1,), (1,)), ((), ())),
        preferred_element_type=jnp.float32,
    ) * _SCALING


def _build_table(lora_At, lora_B):
    return pl.pallas_call(
        _table_body,
        grid=(_V // _VBLK,),
        in_specs=[
            pl.BlockSpec((_VBLK, _R), lambda i: (i, 0)),
            pl.BlockSpec((_D, _R), lambda i: (0, 0)),
        ],
        out_specs=pl.BlockSpec((_VBLK, _D), lambda i: (i, 0)),
        out_shape=jax.ShapeDtypeStruct((_V, _D), jnp.float32),
    )(lora_At, lora_B)


_sc_mesh = plsc.VectorSubcoreMesh(core_axis_name="c", subcore_axis_name="s")


@functools.partial(
    pl.kernel,
    out_type=jax.ShapeDtypeStruct((_NW, _NCHUNK, _RPC, _D), jnp.float32),
    mesh=_sc_mesh,
    scratch_types=[
        pltpu.VMEM((_NCHUNK, _RPC), jnp.int32),
        pltpu.VMEM((_RPC, _D), jnp.float32),
        pltpu.SemaphoreType.DMA,
    ],
)
def _gather(table_hbm, idx_hbm, out_hbm, idx_v, rows_v, sem):
    wid = lax.axis_index("s") * _NC + lax.axis_index("c")
    pltpu.sync_copy(idx_hbm.at[wid], idx_v)

    def body(j, carry):
        pltpu.async_copy(table_hbm.at[idx_v.at[j]], rows_v, sem).wait()
        pltpu.sync_copy(rows_v, out_hbm.at[wid, j])
        return carry

    lax.fori_loop(0, _NCHUNK, body, 0)


def kernel(x, lora_A, lora_B):
    idx = x.astype(jnp.int32).reshape(_NW, _NCHUNK, _RPC)
    table = _build_table(lora_A.T, lora_B)
    out = _gather(table, idx)
    return out.reshape(_B, _L, _D)


# speedup vs baseline: 8.5331x; 1.3717x over previous
import functools

import jax
import jax.numpy as jnp
from jax import lax
from jax.experimental import pallas as pl
from jax.experimental.pallas import tpu as pltpu
from jax.experimental.pallas import tpu_sc as plsc

_SCALING = 16 / 16

_R = 16
_V = 100000
_D = 128
_B = 4096
_L = 200
_NTOK = _B * _L

_NC = 2
_NS = 16
_NW = _NC * _NS
_RPC = 128
_NCHUNK = _NTOK // (_NW * _RPC)

_VBLK = 2000


def _table_body(a_ref, b_ref, o_ref):
    o_ref[...] = lax.dot_general(
        a_ref[...], b_ref[...],
        dimension_numbers=(((1,), (1,)), ((), ())),
        preferred_element_type=jnp.float32,
    ) * _SCALING


def _build_table(lora_At, lora_B):
    return pl.pallas_call(
        _table_body,
        grid=(_V // _VBLK,),
        in_specs=[
            pl.BlockSpec((_VBLK, _R), lambda i: (i, 0)),
            pl.BlockSpec((_D, _R), lambda i: (0, 0)),
        ],
        out_specs=pl.BlockSpec((_VBLK, _D), lambda i: (i, 0)),
        out_shape=jax.ShapeDtypeStruct((_V, _D), jnp.float32),
    )(lora_At, lora_B)


_sc_mesh = plsc.VectorSubcoreMesh(core_axis_name="c", subcore_axis_name="s")


_NBUF = 4


@functools.partial(
    pl.kernel,
    out_type=jax.ShapeDtypeStruct((_NW, _NCHUNK, _RPC, _D), jnp.float32),
    mesh=_sc_mesh,
    scratch_types=[
        pltpu.VMEM((_NCHUNK, _RPC), jnp.int32),
        [pltpu.VMEM((_RPC, _D), jnp.float32) for _ in range(_NBUF)],
        [pltpu.SemaphoreType.DMA for _ in range(_NBUF)],
    ],
)
def _gather(table_hbm, idx_hbm, out_hbm, idx_v, bufs, sems):
    wid = lax.axis_index("s") * _NC + lax.axis_index("c")
    pltpu.sync_copy(idx_hbm.at[wid], idx_v)

    for s in range(_NBUF):
        pltpu.async_copy(table_hbm.at[idx_v.at[s]], bufs[s], sems[s])

    def body(jj, carry):
        j0 = jj * _NBUF
        for s in range(_NBUF):
            j = j0 + s
            pltpu.make_async_copy(
                table_hbm.at[pl.ds(0, _RPC)], bufs[s], sems[s]).wait()
            pltpu.sync_copy(bufs[s], out_hbm.at[wid, j])
            jn = jnp.minimum(j + _NBUF, _NCHUNK - 1)
            pltpu.async_copy(table_hbm.at[idx_v.at[jn]], bufs[s], sems[s])
        return carry

    lax.fori_loop(0, _NCHUNK // _NBUF, body, 0)

    for s in range(_NBUF):
        pltpu.make_async_copy(
            table_hbm.at[pl.ds(0, _RPC)], bufs[s], sems[s]).wait()


def kernel(x, lora_A, lora_B):
    idx = x.astype(jnp.int32).reshape(_NW, _NCHUNK, _RPC)
    table = _build_table(lora_A.T, lora_B)
    out = _gather(table, idx)
    return out.reshape(_B, _L, _D)
